# baseline (speedup 1.0000x reference)
import functools

import jax
import jax.numpy as jnp
from jax import lax
from jax.experimental import pallas as pl
from jax.experimental.pallas import tpu as pltpu
from jax.experimental.pallas import tpu_sc as plsc

NC = 2
NS = 16
NW = NC * NS
CHUNK = 256
NBUF = 5


@functools.lru_cache(maxsize=None)
def _build(b_tot: int, dim: int):
    n_ch = b_tot // (NW * CHUNK)
    assert b_tot == NW * n_ch * CHUNK
    assert n_ch % NBUF == 0
    mesh = plsc.VectorSubcoreMesh(core_axis_name="c", subcore_axis_name="s")

    @functools.partial(
        pl.kernel,
        out_type=jax.ShapeDtypeStruct((b_tot, dim), jnp.float32),
        mesh=mesh,
        compiler_params=pltpu.CompilerParams(use_tc_tiling_on_sc=False),
        scratch_types=(
            [pltpu.VMEM((n_ch, CHUNK), jnp.int32)]
            + [pltpu.VMEM((CHUNK, dim), jnp.float32) for _ in range(NBUF)]
            + [pltpu.SemaphoreType.DMA for _ in range(2 * NBUF)]
        ),
    )
    def gather_kernel(idx_hbm, table_hbm, out_hbm, idx_v, *bufs_and_sems):
        rows = bufs_and_sems[:NBUF]
        gsem = bufs_and_sems[NBUF:2 * NBUF]
        ssem = bufs_and_sems[2 * NBUF:]
        wid = lax.axis_index("s") * NC + lax.axis_index("c")
        row0 = wid * (n_ch * CHUNK)

        pltpu.sync_copy(idx_hbm.at[wid], idx_v)

        def fire_gather(j, b):
            pltpu.async_copy(table_hbm.at[idx_v.at[j]], rows[b], gsem[b])

        def stage(j, b, prefetch):
            pltpu.make_async_copy(
                table_hbm.at[idx_v.at[j]], rows[b], gsem[b]).wait()
            out_slice = out_hbm.at[pl.ds(row0 + j * CHUNK, CHUNK)]
            pltpu.async_copy(rows[b], out_slice, ssem[b])
            if prefetch:
                pltpu.make_async_copy(rows[b], out_slice, ssem[b]).wait()
                fire_gather(j + NBUF, b)

        for b in range(NBUF):
            fire_gather(b, b)

        def outer(o, carry):
            for b in range(NBUF):
                stage(o * NBUF + b, b, True)
            return carry

        lax.fori_loop(0, n_ch // NBUF - 1, outer, 0)
        for b in range(NBUF):
            stage(n_ch - NBUF + b, b, False)
        for b in range(NBUF):
            j = n_ch - NBUF + b
            pltpu.make_async_copy(
                rows[b],
                out_hbm.at[pl.ds(row0 + j * CHUNK, CHUNK)],
                ssem[b]).wait()

    return gather_kernel


def kernel(road_vectors, agent_vectors, emb_weight):
    del road_vectors
    b, n = agent_vectors.shape
    vocab, dim = emb_weight.shape
    b_tot = b * n
    idx = (agent_vectors.astype(jnp.int32) * 2).reshape(
        NW, b_tot // (NW * CHUNK), CHUNK)
    w2 = jnp.reshape(
        jax.lax.optimization_barrier(
            jnp.pad(emb_weight, ((0, 0), (0, 128 - dim)))),
        (2 * vocab, dim))
    out = _build(b_tot, dim)(idx, w2)
    o2 = jax.lax.optimization_barrier(out.reshape(b_tot // 2, 2 * dim))
    return o2.reshape(b, n, dim)


# speedup vs baseline: 1.0155x; 1.0155x over previous
import functools

import jax
import jax.numpy as jnp
from jax import lax
from jax.experimental import pallas as pl
from jax.experimental.pallas import tpu as pltpu
from jax.experimental.pallas import tpu_sc as plsc

NC = 2
NS = 16
NW = NC * NS
CHUNK = 256
NBUF = 5


@functools.lru_cache(maxsize=None)
def _build(b_tot: int, dim: int):
    n_ch = b_tot // (NW * CHUNK)
    assert b_tot == NW * n_ch * CHUNK
    assert n_ch % NBUF == 0
    mesh = plsc.VectorSubcoreMesh(core_axis_name="c", subcore_axis_name="s")

    @functools.partial(
        pl.kernel,
        out_type=jax.ShapeDtypeStruct((b_tot, dim), jnp.float32),
        mesh=mesh,
        compiler_params=pltpu.CompilerParams(use_tc_tiling_on_sc=False),
        scratch_types=(
            [pltpu.VMEM((n_ch, CHUNK), jnp.int32)]
            + [pltpu.VMEM((CHUNK, dim), jnp.float32) for _ in range(NBUF)]
            + [pltpu.SemaphoreType.DMA for _ in range(2 * NBUF)]
        ),
    )
    def gather_kernel(idx_hbm, table_hbm, out_hbm, idx_v, *bufs_and_sems):
        rows = bufs_and_sems[:NBUF]
        gsem = bufs_and_sems[NBUF:2 * NBUF]
        ssem = bufs_and_sems[2 * NBUF:]
        wid = lax.axis_index("s") * NC + lax.axis_index("c")
        row0 = wid * (n_ch * CHUNK)

        pltpu.sync_copy(idx_hbm.at[wid], idx_v)

        def fire_gather(j, b):
            pltpu.async_copy(table_hbm.at[idx_v.at[j]], rows[b], gsem[b])

        def stage(j, b, prefetch):
            pltpu.make_async_copy(
                table_hbm.at[idx_v.at[j]], rows[b], gsem[b]).wait()
            out_slice = out_hbm.at[pl.ds(row0 + j * CHUNK, CHUNK)]
            pltpu.async_copy(rows[b], out_slice, ssem[b])
            if prefetch:
                pltpu.make_async_copy(rows[b], out_slice, ssem[b]).wait()
                fire_gather(j + NBUF, b)

        for b in range(NBUF):
            fire_gather(b, b)

        def outer(o, carry):
            for b in range(NBUF):
                stage(o * NBUF + b, b, True)
            return carry

        lax.fori_loop(0, n_ch // NBUF - 1, outer, 0)
        for b in range(NBUF):
            stage(n_ch - NBUF + b, b, False)
        for b in range(NBUF):
            j = n_ch - NBUF + b
            pltpu.make_async_copy(
                rows[b],
                out_hbm.at[pl.ds(row0 + j * CHUNK, CHUNK)],
                ssem[b]).wait()

    return gather_kernel


def kernel(road_vectors, agent_vectors, emb_weight):
    del road_vectors
    b, n = agent_vectors.shape
    vocab, dim = emb_weight.shape
    b_tot = b * n
    idx = agent_vectors.astype(jnp.int32).reshape(
        NW, b_tot // (NW * CHUNK), CHUNK)
    out = _build(b_tot, dim)(idx, emb_weight)
    return out.reshape(b, n, dim)
